# baseline (speedup 1.0000x reference)
import functools

import jax
import jax.numpy as jnp
from jax.experimental import pallas as pl

N = 10000
HID = 64
HEADS = 8
P = 2


def _elu_block(x_ref, o_ref):
    x = x_ref[...]
    o_ref[...] = jnp.where(x > 0, x, jnp.exp(jnp.minimum(x, 0.0)) - 1.0)


def _pallas_elu(x):
    n, d = x.shape
    blk = 1000
    return pl.pallas_call(
        _elu_block,
        out_shape=jax.ShapeDtypeStruct((n, d), x.dtype),
        grid=(n // blk,),
        in_specs=[pl.BlockSpec((blk, d), lambda i: (i, 0))],
        out_specs=pl.BlockSpec((blk, d), lambda i: (i, 0)),
    )(x)


def _elu(x):
    return jnp.where(x > 0, x, jnp.expm1(x))


def _metapath(h, idx, attn1_W, attn2):
    dst = idx[:, -1]
    edata = h[idx]
    hidden = jnp.mean(edata, axis=1)
    eft = jnp.broadcast_to(hidden[:, None, :], (hidden.shape[0], HEADS, HID))
    a1 = h[dst] @ attn1_W
    a2 = jnp.sum(eft * attn2[None, :, :], axis=-1)
    a = jax.nn.leaky_relu(a1 + a2, 0.01)
    amax = jax.ops.segment_max(a, dst, num_segments=N)
    amax = jnp.where(jnp.isfinite(amax), amax, 0.0)
    ex = jnp.exp(a - amax[dst])
    denom = jax.ops.segment_sum(ex, dst, num_segments=N)
    att = ex / (denom[dst] + 1e-9)
    ret = jax.ops.segment_sum(att[:, :, None] * eft, dst, num_segments=N)
    return _pallas_elu(ret.reshape(N, HEADS * HID))


def _nc_layer(h, idxs, p):
    outs = [_metapath(h, idxs[i], p["attn1"][i], p["attn2"][i]) for i in range(P)]
    betas = []
    for hp in outs:
        fc1 = jnp.tanh(hp @ p["Ws1"] + p["bs1"])
        betas.append(jnp.mean(fc1, axis=0) @ p["Ws2"])
    beta = jax.nn.softmax(jnp.concatenate(betas))
    hcat = sum(beta[i] * outs[i] for i in range(P))
    return hcat @ p["Wfc"] + p["bfc"], hcat


def kernel(features, type_mask, edge_metapath_indices_0, edge_metapath_indices_1, target_node_indices, params):
    idxs = [edge_metapath_indices_0, edge_metapath_indices_1]
    h = features @ params["W_feat"] + params["b_feat"]
    h_fc, _ = _nc_layer(h, idxs, params["layer0"])
    h = _elu(h_fc)
    logits, h_last = _nc_layer(h, idxs, params["layer1"])
    return logits[target_node_indices], h_last[target_node_indices]


# speedup vs baseline: 15.7563x; 15.7563x over previous
import functools

import jax
import jax.numpy as jnp
from jax import lax
from jax.experimental import pallas as pl
from jax.experimental.pallas import tpu as pltpu
from jax.experimental.pallas import tpu_sc as plsc

N = 10000
HID = 64
HEADS = 8
P = 2
E = 160000
L = 3
FC = 16
NCHUNK = HID // FC
EB = 128
NBLK = E // EB
NW = 32
ZB = 80
NZB = N // ZB


def _permute(v, idx):
    dn = lax.GatherDimensionNumbers(
        offset_dims=(), collapsed_slice_dims=(0,), start_index_map=(0,))
    return lax.gather(v, idx[:, None], dn, slice_sizes=(1,),
                      mode=lax.GatherScatterMode.PROMISE_IN_BOUNDS)


def _make_sc_kernel(trow, row, with_outer):
    mesh = plsc.VectorSubcoreMesh(core_axis_name="c", subcore_axis_name="s")

    @functools.partial(
        pl.kernel,
        mesh=mesh,
        compiler_params=pltpu.CompilerParams(use_tc_tiling_on_sc=False),
        out_type=jax.ShapeDtypeStruct((2, N, row), jnp.float32),
        scratch_types=[
            pltpu.VMEM((EB,), jnp.int32),
            pltpu.VMEM((EB,), jnp.int32),
            pltpu.VMEM((EB,), jnp.int32),
            pltpu.VMEM((EB, trow), jnp.float32),
            pltpu.VMEM((EB, trow), jnp.float32),
            pltpu.VMEM((EB, trow), jnp.float32),
            pltpu.VMEM((EB, row), jnp.float32),
            pltpu.VMEM_SHARED((N, row), jnp.float32),
            pltpu.SemaphoreType.DMA,
        ],
    )
    def body(idx0_hbm, idx1_hbm, idx2_hbm, zeros_hbm, tab_hbm, out_hbm,
             i0_v, i1_v, i2_v, r0_v, r1_v, r2_v, contrib_v, acc_sh, gsem):
        c = lax.axis_index("c")
        s = lax.axis_index("s")
        wid = s * 2 + c
        lane = lax.iota(jnp.int32, 16)
        rot8 = (lane + 8) & 15
        third = jnp.float32(1.0 / 3.0)
        boff = 16 if with_outer else 0

        nz = (NZB - s + 15) // 16

        def zblk(t, carry):
            z = s + t * 16
            pltpu.sync_copy(zeros_hbm, acc_sh.at[pl.ds(z * ZB, ZB), :])
            return carry
        lax.fori_loop(0, nz, zblk, 0)
        plsc.subcore_barrier()

        nt = (NBLK - wid + NW - 1) // NW

        def eblk(t, carry):
            base = (wid + t * NW) * EB
            pltpu.sync_copy(idx0_hbm.at[pl.ds(base, EB)], i0_v)
            pltpu.sync_copy(idx1_hbm.at[pl.ds(base, EB)], i1_v)
            pltpu.sync_copy(idx2_hbm.at[pl.ds(base, EB)], i2_v)
            cp0 = pltpu.async_copy(tab_hbm.at[i0_v], r0_v, gsem)
            cp1 = pltpu.async_copy(tab_hbm.at[i1_v], r1_v, gsem)
            cp2 = pltpu.async_copy(tab_hbm.at[i2_v], r2_v, gsem)
            cp0.wait()
            cp1.wait()
            cp2.wait()

            def edge(eo, carry2):
                for k in range(4):
                    e = eo * 4 + k
                    b0 = r0_v[e, boff:boff + 16]
                    b1 = r1_v[e, boff:boff + 16]
                    b2 = r2_v[e, boff:boff + 16]
                    sb = b0 + b1 + b2
                    amix = b2 + _permute(sb, rot8) * third
                    av = jnp.where(amix > 0, amix, amix * jnp.float32(0.01))
                    av = jnp.minimum(av, jnp.float32(75.0))
                    ex = jnp.exp(av)
                    if with_outer:
                        a0 = r0_v[e, 0:16]
                        a1 = r1_v[e, 0:16]
                        a2 = r2_v[e, 0:16]
                        hid16 = (a0 + a1 + a2) * third
                        for h in range(HEADS):
                            bh = _permute(ex, jnp.full((16,), h, jnp.int32))
                            contrib_v[e, pl.ds(h * 16, 16)] = bh * hid16
                    else:
                        exm = jnp.where(lane < 8, ex, jnp.float32(0.0))
                        contrib_v[e, pl.ds(0, 16)] = exm
                return carry2
            lax.fori_loop(0, EB // 4, edge, 0)
            pltpu.sync_copy(contrib_v, acc_sh.at[i2_v], add=True)
            return carry
        lax.fori_loop(0, nt, eblk, 0)
        plsc.subcore_barrier()

        def dblk(t, carry):
            z = s + t * 16
            pltpu.sync_copy(acc_sh.at[pl.ds(z * ZB, ZB), :],
                            out_hbm.at[c, pl.ds(z * ZB, ZB), :])
            return carry
        lax.fori_loop(0, nz, dblk, 0)

    return body


_mp_chunk = _make_sc_kernel(32, 8 * FC, True)
_mp_denom = _make_sc_kernel(16, 16, False)


def _elu(x):
    return jnp.where(x > 0, x, jnp.expm1(x))


def _sc_metapath(h, idxT, attn1_W, attn2):
    i0, i1, i2 = idxT
    z16 = jnp.zeros((ZB, 16), jnp.float32)
    z128 = jnp.zeros((ZB, 128), jnp.float32)
    ha1 = h @ attn1_W
    ha2 = h @ attn2.T
    tab2 = jnp.concatenate([ha1, ha2], axis=1)
    dout = _mp_denom(i0, i1, i2, z16, tab2)
    denom = (dout[0] + dout[1])[:, :8]
    outs = []
    for ci in range(NCHUNK):
        tab = jnp.concatenate([h[:, ci * FC:(ci + 1) * FC], ha1, ha2], axis=1)
        out = _mp_chunk(i0, i1, i2, z128, tab)
        acc = out[0] + out[1]
        outs.append(acc.reshape(N, HEADS, FC))
    ret = jnp.concatenate(outs, axis=-1)
    ret = ret / (denom[:, :, None] + 1e-9)
    return _elu(ret.reshape(N, HEADS * HID))


def _nc_layer(h, idxTs, p):
    outs = [_sc_metapath(h, idxTs[i], p["attn1"][i], p["attn2"][i])
            for i in range(P)]
    betas = []
    for hp in outs:
        fc1 = jnp.tanh(hp @ p["Ws1"] + p["bs1"])
        betas.append(jnp.mean(fc1, axis=0) @ p["Ws2"])
    beta = jax.nn.softmax(jnp.concatenate(betas))
    hcat = sum(beta[i] * outs[i] for i in range(P))
    return hcat @ p["Wfc"] + p["bfc"], hcat


def kernel(features, type_mask, edge_metapath_indices_0,
           edge_metapath_indices_1, target_node_indices, params):
    idxTs = [tuple(m[:, l] for l in range(L))
             for m in (edge_metapath_indices_0, edge_metapath_indices_1)]
    h = features @ params["W_feat"] + params["b_feat"]
    h_fc, _ = _nc_layer(h, idxTs, params["layer0"])
    h = _elu(h_fc)
    logits, h_last = _nc_layer(h, idxTs, params["layer1"])
    return logits[target_node_indices], h_last[target_node_indices]
